# baseline (speedup 1.0000x reference)
import jax
import jax.numpy as jnp
from jax import lax
from jax.experimental import pallas as pl
from jax.experimental.pallas import tpu as pltpu
from jax.experimental.pallas import tpu_sc as plsc

N = 10000
E = 320000
H = 128
HALF = 64
NC = 2
NS = 16
EB = 40
ES = E // NS
NCHUNK = ES // EB
NPAIR = NCHUNK // 2
ZB = 40
NZCHUNK = N // ZB
NZROUND = (NZCHUNK + NS - 1) // NS
RB = 200
NRCHUNK = N // RB
NRROUND = (NRCHUNK + NS - 1) // NS

_f32 = jnp.float32


def _sc_edge_body(xin_hbm, ea_hbm, src_hbm, dst_hbm, t_hbm, u_hbm, acc_hbm,
                  acc_sh, srcb0, srcb1, dstb0, dstb1, xsb0, xsb1, eab0, eab1,
                  valb0, valb1, tb, ub, ssrc0, ssrc1, sdat0, sdat1,
                  sdst0, sdst1, sscat0, sscat1):
    srcb = (srcb0, srcb1)
    dstb = (dstb0, dstb1)
    xsb = (xsb0, xsb1)
    eab = (eab0, eab1)
    valb = (valb0, valb1)
    ssrc = (ssrc0, ssrc1)
    sdat = (sdat0, sdat1)
    sdst = (sdst0, sdst1)
    sscat = (sscat0, sscat1)
    c = lax.axis_index("c")
    s = lax.axis_index("s")

    pltpu.sync_copy(t_hbm, tb)
    pltpu.sync_copy(u_hbm, ub)
    tv = tb[...]
    uv = ub[...]

    zero16 = jnp.zeros((16,), _f32)

    def _zrow(r, _):
        for q in range(2 * HALF // 16):
            valb0[r, pl.ds(16 * q, 16)] = zero16
        return 0

    lax.fori_loop(0, ZB, _zrow, 0)
    for k in range(NZROUND):
        cid = s + NS * k

        @pl.when(cid < NZCHUNK)
        def _():
            pltpu.sync_copy(valb0, acc_sh.at[pl.ds(cid * ZB, ZB)])
    plsc.subcore_barrier()

    base_e = s * ES
    c64 = c * HALF

    def issue_src(e0, p):
        pltpu.async_copy(src_hbm.at[pl.ds(e0, EB)], srcb[p], ssrc[p])

    def wait_src(p):
        pltpu.make_async_copy(src_hbm.at[pl.ds(0, EB)], srcb[p], ssrc[p]).wait()

    def issue_dat(e0, p):
        pltpu.async_copy(xin_hbm.at[srcb[p]], xsb[p], sdat[p])
        pltpu.async_copy(ea_hbm.at[pl.ds(e0, EB)], eab[p], sdat[p])

    def wait_dat(p):
        pltpu.make_async_copy(xin_hbm.at[srcb[p]], xsb[p], sdat[p]).wait()
        pltpu.make_async_copy(ea_hbm.at[pl.ds(0, EB)], eab[p], sdat[p]).wait()

    def wait_scat(p):
        pltpu.make_async_copy(valb[p], acc_sh.at[dstb[p]], sscat[p]).wait()

    def step(j, p):
        q = 1 - p

        @pl.when(j + 1 < NCHUNK)
        def _():
            wait_src(q)
            issue_dat(base_e + (j + 1) * EB, q)

        wait_dat(p)

        @pl.when(j + 2 < NCHUNK)
        def _():
            issue_src(base_e + (j + 2) * EB, p)

        @pl.when(j >= 2)
        def _():
            wait_scat(p)

        pltpu.async_copy(dst_hbm.at[pl.ds(base_e + j * EB, EB)], dstb[p],
                         sdst[p])

        @plsc.parallel_loop(0, EB, unroll=8)
        def _edge(e):
            for qq in range(HALF // 16):
                xq = xsb[p][e, pl.ds(c64 + 16 * qq, 16)]
                aq = eab[p][e, pl.ds(c64 + 16 * qq, 16)]
                msg = jnp.maximum(xq + aq, 0.0) + 1e-7
                pw = jnp.exp(msg * tv - uv)
                valb[p][e, pl.ds(16 * qq, 16)] = msg * pw
                valb[p][e, pl.ds(HALF + 16 * qq, 16)] = pw

        pltpu.make_async_copy(dst_hbm.at[pl.ds(0, EB)], dstb[p], sdst[p]).wait()
        pltpu.async_copy(valb[p], acc_sh.at[dstb[p]], sscat[p], add=True)

    issue_src(base_e, 0)
    issue_src(base_e + EB, 1)
    wait_src(0)
    issue_dat(base_e, 0)

    def _pair(i, _):
        step(2 * i, 0)
        step(2 * i + 1, 1)
        return 0

    lax.fori_loop(0, NPAIR, _pair, 0)
    wait_scat(0)
    wait_scat(1)
    plsc.subcore_barrier()

    cN = c * N
    for k in range(NRROUND):
        cid = s + NS * k

        @pl.when(cid < NRCHUNK)
        def _():
            r0 = cid * RB
            pltpu.sync_copy(acc_sh.at[pl.ds(r0, RB)],
                            acc_hbm.at[pl.ds(cN + r0, RB)])


_sc_edge = pl.kernel(
    _sc_edge_body,
    out_type=jax.ShapeDtypeStruct((NC * N, 2 * HALF), _f32),
    mesh=plsc.VectorSubcoreMesh(core_axis_name="c", subcore_axis_name="s"),
    scratch_types=[
        pltpu.VMEM_SHARED((N, 2 * HALF), _f32),
        pltpu.VMEM((EB,), jnp.int32),
        pltpu.VMEM((EB,), jnp.int32),
        pltpu.VMEM((EB,), jnp.int32),
        pltpu.VMEM((EB,), jnp.int32),
        pltpu.VMEM((EB, H), _f32),
        pltpu.VMEM((EB, H), _f32),
        pltpu.VMEM((EB, H), _f32),
        pltpu.VMEM((EB, H), _f32),
        pltpu.VMEM((EB, 2 * HALF), _f32),
        pltpu.VMEM((EB, 2 * HALF), _f32),
        pltpu.VMEM((16,), _f32),
        pltpu.VMEM((16,), _f32),
        pltpu.SemaphoreType.DMA,
        pltpu.SemaphoreType.DMA,
        pltpu.SemaphoreType.DMA,
        pltpu.SemaphoreType.DMA,
        pltpu.SemaphoreType.DMA,
        pltpu.SemaphoreType.DMA,
        pltpu.SemaphoreType.DMA,
        pltpu.SemaphoreType.DMA,
    ],
)


def _acc_max(mx_ref, m):
    i = pl.program_id(0)

    @pl.when(i == 0)
    def _():
        mx_ref[0, 0] = m

    @pl.when(i > 0)
    def _():
        mx_ref[0, 0] = jnp.maximum(mx_ref[0, 0], m)


def _k_edge_enc(ea_ref, w_ref, b_ref, out_ref, mx_ref):
    ea = jnp.dot(ea_ref[...], w_ref[...], preferred_element_type=_f32) + b_ref[...]
    out_ref[...] = ea
    _acc_max(mx_ref, jnp.max(ea))


def _k_node_enc(x_ref, w_ref, b_ref, out_ref, mx_ref):
    y = jnp.dot(x_ref[...], w_ref[...], preferred_element_type=_f32) + b_ref[...]
    out_ref[...] = y
    _acc_max(mx_ref, jnp.max(y))


def _ln_rows(x, g, b):
    mu = jnp.mean(x, axis=-1, keepdims=True)
    xc = x - mu
    var = jnp.mean(xc * xc, axis=-1, keepdims=True)
    return xc * jax.lax.rsqrt(var + 1e-5) * g + b


def _k_norm_act(x_ref, g_ref, b_ref, out_ref, mx_ref):
    y = jnp.maximum(_ln_rows(x_ref[...], g_ref[...], b_ref[...]), 0.0)
    out_ref[...] = y
    _acc_max(mx_ref, jnp.max(y))


def _k_mlp(xin_ref, acc_ref, xprev_ref, w1_ref, b1_ref, g1_ref, be1_ref,
           w2_ref, b2_ref, out_ref):
    a0 = acc_ref[0]
    a1 = acc_ref[1]
    lo = jnp.where(a0[:, HALF:] > 0.0, a0[:, :HALF] / a0[:, HALF:], 0.0)
    hi = jnp.where(a1[:, HALF:] > 0.0, a1[:, :HALF] / a1[:, HALF:], 0.0)
    h = xin_ref[...] + jnp.concatenate([lo, hi], axis=-1)
    z = jnp.dot(h, w1_ref[...], preferred_element_type=_f32) + b1_ref[...]
    z = jnp.maximum(_ln_rows(z, g1_ref[...], be1_ref[...]), 0.0)
    z = jnp.dot(z, w2_ref[...], preferred_element_type=_f32) + b2_ref[...]
    out_ref[...] = xprev_ref[...] + z


def _k_final(x_ref, g_ref, b_ref, w_ref, bo_ref, out_ref):
    y = jnp.maximum(_ln_rows(x_ref[...], g_ref[...], b_ref[...]), 0.0)
    out_ref[...] = jnp.dot(y, w_ref[...], preferred_element_type=_f32) + bo_ref[...]


def _full(shape):
    nd = len(shape)
    return pl.BlockSpec(shape, lambda i, _nd=nd: (0,) * _nd)


def _enc_max_call(body, x, w, b, bn):
    n = x.shape[0]
    return pl.pallas_call(
        body,
        grid=(n // bn,),
        in_specs=[
            pl.BlockSpec((bn, x.shape[1]), lambda i: (i, 0)),
            _full(w.shape),
            _full(b.shape),
        ],
        out_specs=[
            pl.BlockSpec((bn, H), lambda i: (i, 0)),
            pl.BlockSpec(memory_space=pltpu.SMEM),
        ],
        out_shape=[
            jax.ShapeDtypeStruct((n, H), _f32),
            jax.ShapeDtypeStruct((1, 1), _f32),
        ],
    )(x, w, b)


BN = 2000
BE_ENC = 4000


def kernel(x, edge_index, edge_attr, params):
    src = edge_index[0]
    dst = edge_index[1]

    We, be = params['edge_enc']
    ea, mx_ea = _enc_max_call(_k_edge_enc, edge_attr, We, be.reshape(1, H),
                              BE_ENC)

    Wn, bn_ = params['node_enc']
    xin, mx_x = _enc_max_call(_k_node_enc, x, Wn, bn_.reshape(1, H), BN)

    mlp_call = pl.pallas_call(
        _k_mlp,
        grid=(N // BN,),
        in_specs=[
            pl.BlockSpec((BN, H), lambda i: (i, 0)),
            pl.BlockSpec((2, BN, H), lambda i: (0, i, 0)),
            pl.BlockSpec((BN, H), lambda i: (i, 0)),
            _full((H, 2 * H)), _full((1, 2 * H)), _full((1, 2 * H)),
            _full((1, 2 * H)), _full((2 * H, H)), _full((1, H)),
        ],
        out_specs=pl.BlockSpec((BN, H), lambda i: (i, 0)),
        out_shape=jax.ShapeDtypeStruct((N, H), _f32),
    )

    x_run = jnp.zeros((N, H), _f32)
    for li, lp in enumerate(params['layers']):
        if li > 0:
            g, bb = lp['norm']
            xin, mx_x = _enc_max_call(
                _k_norm_act, x_run, g.reshape(1, H), bb.reshape(1, H), BN)
        t = lp['t']
        u = t * (jnp.maximum(mx_x[0, 0] + mx_ea[0, 0], 0.0) + 1e-7)
        t16 = jnp.broadcast_to(t.astype(_f32), (16,))
        u16 = jnp.broadcast_to(u.astype(_f32), (16,))
        acc_flat = _sc_edge(xin, ea, src, dst, t16, u16)
        acc2 = acc_flat.reshape(NC, N, 2 * HALF)
        W1, b1 = lp['mlp_w1']
        g1, be1 = lp['mlp_ln']
        W2, b2 = lp['mlp_w2']
        x_run = mlp_call(xin, acc2, x_run, W1, b1.reshape(1, 2 * H),
                         g1.reshape(1, 2 * H), be1.reshape(1, 2 * H),
                         W2, b2.reshape(1, H))

    g0, b0 = params['layers'][0]['norm']
    Wo, bo = params['lin_out']
    out = pl.pallas_call(
        _k_final,
        grid=(N // BN,),
        in_specs=[
            pl.BlockSpec((BN, H), lambda i: (i, 0)),
            _full((1, H)), _full((1, H)), _full((H, H)), _full((1, H)),
        ],
        out_specs=pl.BlockSpec((BN, H), lambda i: (i, 0)),
        out_shape=jax.ShapeDtypeStruct((N, H), _f32),
    )(x_run, g0.reshape(1, H), b0.reshape(1, H), Wo, bo.reshape(1, H))
    return out


# speedup vs baseline: 1.2650x; 1.2650x over previous
import jax
import jax.numpy as jnp
from jax import lax
from jax.experimental import pallas as pl
from jax.experimental.pallas import tpu as pltpu
from jax.experimental.pallas import tpu_sc as plsc

N = 10000
E = 320000
H = 128
HALF = 64
NC = 2
NS = 16
EB = 64
TCH = E // EB
NSTEP = (TCH + NS - 1) // NS + ((TCH + NS - 1) // NS) % 2
NPAIR = NSTEP // 2
ZB = 40
NZCHUNK = N // ZB
NZROUND = (NZCHUNK + NS - 1) // NS
RB = 200
NRCHUNK = N // RB
NRROUND = (NRCHUNK + NS - 1) // NS

_f32 = jnp.float32


def _sc_edge_body(xin_hbm, ea_hbm, src_hbm, dst_hbm, t_hbm, u_hbm, acc_hbm,
                  acc_sh, srcb0, srcb1, dstb0, dstb1, xsb0, xsb1, eab0, eab1,
                  valb0, valb1, tb, ub, ssrc0, ssrc1, sdat0, sdat1,
                  sdst0, sdst1, sscat0, sscat1):
    srcb = (srcb0, srcb1)
    dstb = (dstb0, dstb1)
    xsb = (xsb0, xsb1)
    eab = (eab0, eab1)
    valb = (valb0, valb1)
    ssrc = (ssrc0, ssrc1)
    sdat = (sdat0, sdat1)
    sdst = (sdst0, sdst1)
    sscat = (sscat0, sscat1)
    c = lax.axis_index("c")
    s = lax.axis_index("s")

    pltpu.sync_copy(t_hbm, tb)
    pltpu.sync_copy(u_hbm, ub)
    tv = tb[...]
    uv = ub[...]

    zero16 = jnp.zeros((16,), _f32)

    def _zrow(r, _):
        for q in range(2 * HALF // 16):
            valb0[r, pl.ds(16 * q, 16)] = zero16
        return 0

    lax.fori_loop(0, ZB, _zrow, 0)
    for k in range(NZROUND):
        cid = s + NS * k

        @pl.when(cid < NZCHUNK)
        def _():
            pltpu.sync_copy(valb0.at[pl.ds(0, ZB)],
                            acc_sh.at[pl.ds(cid * ZB, ZB)])
    plsc.subcore_barrier()

    c64 = c * HALF
    cEh = c * (E // 2)

    def _e0(jj):
        return (s + NS * jj) * EB

    def _valid(jj):
        return s + NS * jj < TCH

    def issue_src(jj, p):
        pltpu.async_copy(src_hbm.at[pl.ds(_e0(jj), EB)], srcb[p], ssrc[p])

    def wait_src(p):
        pltpu.make_async_copy(src_hbm.at[pl.ds(0, EB)], srcb[p], ssrc[p]).wait()

    def issue_dat(jj, p):
        pltpu.async_copy(xin_hbm.at[srcb[p]], xsb[p], sdat[p])
        pltpu.async_copy(
            ea_hbm.at[pl.ds(cEh + (s + NS * jj) * (EB // 2), EB // 2)],
            eab[p], sdat[p])

    def wait_dat(p):
        pltpu.make_async_copy(xin_hbm.at[srcb[p]], xsb[p], sdat[p]).wait()
        pltpu.make_async_copy(ea_hbm.at[pl.ds(0, EB // 2)], eab[p],
                              sdat[p]).wait()

    def wait_scat(p):
        pltpu.make_async_copy(valb[p], acc_sh.at[dstb[p]], sscat[p]).wait()

    def step(jj, p):
        q = 1 - p

        @pl.when(_valid(jj + 1))
        def _():
            wait_src(q)
            issue_dat(jj + 1, q)

        @pl.when(_valid(jj))
        def _():
            wait_dat(p)

            @pl.when(_valid(jj + 2))
            def _():
                issue_src(jj + 2, p)

            @pl.when(jj >= 2)
            def _():
                wait_scat(p)

            pltpu.async_copy(dst_hbm.at[pl.ds(_e0(jj), EB)], dstb[p], sdst[p])

            @plsc.parallel_loop(0, EB // 2, unroll=2)
            def _pair_edges(k):
                for par in range(2):
                    e = 2 * k + par
                    for qq in range(HALF // 16):
                        xq = xsb[p][e, pl.ds(c64 + 16 * qq, 16)]
                        aq = eab[p][k, pl.ds(par * HALF + 16 * qq, 16)]
                        msg = jnp.maximum(xq + aq, 0.0) + 1e-7
                        pw = jnp.exp(msg * tv - uv)
                        valb[p][e, pl.ds(16 * qq, 16)] = msg * pw
                        valb[p][e, pl.ds(HALF + 16 * qq, 16)] = pw

            pltpu.make_async_copy(dst_hbm.at[pl.ds(0, EB)], dstb[p],
                                  sdst[p]).wait()
            pltpu.async_copy(valb[p], acc_sh.at[dstb[p]], sscat[p], add=True)

    issue_src(0, 0)
    issue_src(1, 1)
    wait_src(0)
    issue_dat(0, 0)

    def _pair(i, _):
        step(2 * i, 0)
        step(2 * i + 1, 1)
        return 0

    lax.fori_loop(0, NPAIR, _pair, 0)
    wait_scat(0)
    wait_scat(1)
    plsc.subcore_barrier()

    cN = c * N
    for k in range(NRROUND):
        cid = s + NS * k

        @pl.when(cid < NRCHUNK)
        def _():
            r0 = cid * RB
            pltpu.sync_copy(acc_sh.at[pl.ds(r0, RB)],
                            acc_hbm.at[pl.ds(cN + r0, RB)])


_sc_edge = pl.kernel(
    _sc_edge_body,
    out_type=jax.ShapeDtypeStruct((NC * N, 2 * HALF), _f32),
    mesh=plsc.VectorSubcoreMesh(core_axis_name="c", subcore_axis_name="s"),
    scratch_types=[
        pltpu.VMEM_SHARED((N, 2 * HALF), _f32),
        pltpu.VMEM((EB,), jnp.int32),
        pltpu.VMEM((EB,), jnp.int32),
        pltpu.VMEM((EB,), jnp.int32),
        pltpu.VMEM((EB,), jnp.int32),
        pltpu.VMEM((EB, H), _f32),
        pltpu.VMEM((EB, H), _f32),
        pltpu.VMEM((EB // 2, H), _f32),
        pltpu.VMEM((EB // 2, H), _f32),
        pltpu.VMEM((EB, 2 * HALF), _f32),
        pltpu.VMEM((EB, 2 * HALF), _f32),
        pltpu.VMEM((16,), _f32),
        pltpu.VMEM((16,), _f32),
        pltpu.SemaphoreType.DMA,
        pltpu.SemaphoreType.DMA,
        pltpu.SemaphoreType.DMA,
        pltpu.SemaphoreType.DMA,
        pltpu.SemaphoreType.DMA,
        pltpu.SemaphoreType.DMA,
        pltpu.SemaphoreType.DMA,
        pltpu.SemaphoreType.DMA,
    ],
)


def _acc_max(mx_ref, m):
    i = pl.program_id(0)

    @pl.when(i == 0)
    def _():
        mx_ref[0, 0] = m

    @pl.when(i > 0)
    def _():
        mx_ref[0, 0] = jnp.maximum(mx_ref[0, 0], m)


def _k_edge_enc(a2_ref, wlo_ref, whi_ref, blo_ref, bhi_ref, out_ref, mx_ref):
    a2 = a2_ref[...]
    lo = jnp.dot(a2, wlo_ref[...], preferred_element_type=_f32) + blo_ref[...]
    hi = jnp.dot(a2, whi_ref[...], preferred_element_type=_f32) + bhi_ref[...]
    out_ref[0] = lo
    out_ref[1] = hi
    _acc_max(mx_ref, jnp.maximum(jnp.max(lo), jnp.max(hi)))


def _k_node_enc(x_ref, w_ref, b_ref, out_ref, mx_ref):
    y = jnp.dot(x_ref[...], w_ref[...], preferred_element_type=_f32) + b_ref[...]
    out_ref[...] = y
    _acc_max(mx_ref, jnp.max(y))


def _ln_rows(x, g, b):
    mu = jnp.mean(x, axis=-1, keepdims=True)
    xc = x - mu
    var = jnp.mean(xc * xc, axis=-1, keepdims=True)
    return xc * jax.lax.rsqrt(var + 1e-5) * g + b


def _k_norm_act(x_ref, g_ref, b_ref, out_ref, mx_ref):
    y = jnp.maximum(_ln_rows(x_ref[...], g_ref[...], b_ref[...]), 0.0)
    out_ref[...] = y
    _acc_max(mx_ref, jnp.max(y))


def _k_mlp(xin_ref, acc_ref, xprev_ref, w1_ref, b1_ref, g1_ref, be1_ref,
           w2_ref, b2_ref, out_ref):
    a0 = acc_ref[0]
    a1 = acc_ref[1]
    lo = jnp.where(a0[:, HALF:] > 0.0, a0[:, :HALF] / a0[:, HALF:], 0.0)
    hi = jnp.where(a1[:, HALF:] > 0.0, a1[:, :HALF] / a1[:, HALF:], 0.0)
    h = xin_ref[...] + jnp.concatenate([lo, hi], axis=-1)
    z = jnp.dot(h, w1_ref[...], preferred_element_type=_f32) + b1_ref[...]
    z = jnp.maximum(_ln_rows(z, g1_ref[...], be1_ref[...]), 0.0)
    z = jnp.dot(z, w2_ref[...], preferred_element_type=_f32) + b2_ref[...]
    out_ref[...] = xprev_ref[...] + z


def _k_final(x_ref, g_ref, b_ref, w_ref, bo_ref, out_ref):
    y = jnp.maximum(_ln_rows(x_ref[...], g_ref[...], b_ref[...]), 0.0)
    out_ref[...] = jnp.dot(y, w_ref[...], preferred_element_type=_f32) + bo_ref[...]


def _full(shape):
    nd = len(shape)
    return pl.BlockSpec(shape, lambda i, _nd=nd: (0,) * _nd)


def _enc_max_call(body, x, w, b, bn, packed=False):
    n = x.shape[0]
    if packed:
        out_spec = pl.BlockSpec((2, bn // 2, H), lambda i: (0, i, 0))
        out_sds = jax.ShapeDtypeStruct((2, n // 2, H), _f32)
    else:
        out_spec = pl.BlockSpec((bn, H), lambda i: (i, 0))
        out_sds = jax.ShapeDtypeStruct((n, H), _f32)
    return pl.pallas_call(
        body,
        grid=(n // bn,),
        in_specs=[
            pl.BlockSpec((bn, x.shape[1]), lambda i: (i, 0)),
            _full(w.shape),
            _full(b.shape),
        ],
        out_specs=[
            out_spec,
            pl.BlockSpec(memory_space=pltpu.SMEM),
        ],
        out_shape=[
            out_sds,
            jax.ShapeDtypeStruct((1, 1), _f32),
        ],
    )(x, w, b)


BN = 2000
BE_ENC = 4000


def kernel(x, edge_index, edge_attr, params):
    src = edge_index[0]
    dst = edge_index[1]

    We, be = params['edge_enc']
    D_E = edge_attr.shape[1]
    attr2 = edge_attr.reshape(E // 2, 2 * D_E)
    zpad = jnp.zeros((D_E, HALF), _f32)
    wlo = jnp.concatenate(
        [jnp.concatenate([We[:, :HALF], zpad], axis=1),
         jnp.concatenate([zpad, We[:, :HALF]], axis=1)], axis=0)
    whi = jnp.concatenate(
        [jnp.concatenate([We[:, HALF:], zpad], axis=1),
         jnp.concatenate([zpad, We[:, HALF:]], axis=1)], axis=0)
    blo = jnp.concatenate([be[:HALF], be[:HALF]]).reshape(1, H)
    bhi = jnp.concatenate([be[HALF:], be[HALF:]]).reshape(1, H)
    BE2 = 2000
    ea_pack, mx_ea = pl.pallas_call(
        _k_edge_enc,
        grid=(E // 2 // BE2,),
        in_specs=[
            pl.BlockSpec((BE2, 2 * D_E), lambda i: (i, 0)),
            _full((2 * D_E, H)), _full((2 * D_E, H)),
            _full((1, H)), _full((1, H)),
        ],
        out_specs=[
            pl.BlockSpec((2, BE2, H), lambda i: (0, i, 0)),
            pl.BlockSpec(memory_space=pltpu.SMEM),
        ],
        out_shape=[
            jax.ShapeDtypeStruct((2, E // 2, H), _f32),
            jax.ShapeDtypeStruct((1, 1), _f32),
        ],
    )(attr2, wlo, whi, blo, bhi)
    ea = ea_pack.reshape(E, H)

    Wn, bn_ = params['node_enc']
    xin, mx_x = _enc_max_call(_k_node_enc, x, Wn, bn_.reshape(1, H), BN)

    mlp_call = pl.pallas_call(
        _k_mlp,
        grid=(N // BN,),
        in_specs=[
            pl.BlockSpec((BN, H), lambda i: (i, 0)),
            pl.BlockSpec((2, BN, H), lambda i: (0, i, 0)),
            pl.BlockSpec((BN, H), lambda i: (i, 0)),
            _full((H, 2 * H)), _full((1, 2 * H)), _full((1, 2 * H)),
            _full((1, 2 * H)), _full((2 * H, H)), _full((1, H)),
        ],
        out_specs=pl.BlockSpec((BN, H), lambda i: (i, 0)),
        out_shape=jax.ShapeDtypeStruct((N, H), _f32),
    )

    x_run = jnp.zeros((N, H), _f32)
    for li, lp in enumerate(params['layers']):
        if li > 0:
            g, bb = lp['norm']
            xin, mx_x = _enc_max_call(
                _k_norm_act, x_run, g.reshape(1, H), bb.reshape(1, H), BN)
        t = lp['t']
        u = t * (jnp.maximum(mx_x[0, 0] + mx_ea[0, 0], 0.0) + 1e-7)
        t16 = jnp.broadcast_to(t.astype(_f32), (16,))
        u16 = jnp.broadcast_to(u.astype(_f32), (16,))
        acc_flat = _sc_edge(xin, ea, src, dst, t16, u16)
        acc2 = acc_flat.reshape(NC, N, 2 * HALF)
        W1, b1 = lp['mlp_w1']
        g1, be1 = lp['mlp_ln']
        W2, b2 = lp['mlp_w2']
        x_run = mlp_call(xin, acc2, x_run, W1, b1.reshape(1, 2 * H),
                         g1.reshape(1, 2 * H), be1.reshape(1, 2 * H),
                         W2, b2.reshape(1, H))

    g0, b0 = params['layers'][0]['norm']
    Wo, bo = params['lin_out']
    out = pl.pallas_call(
        _k_final,
        grid=(N // BN,),
        in_specs=[
            pl.BlockSpec((BN, H), lambda i: (i, 0)),
            _full((1, H)), _full((1, H)), _full((H, H)), _full((1, H)),
        ],
        out_specs=pl.BlockSpec((BN, H), lambda i: (i, 0)),
        out_shape=jax.ShapeDtypeStruct((N, H), _f32),
    )(x_run, g0.reshape(1, H), b0.reshape(1, H), Wo, bo.reshape(1, H))
    return out


# speedup vs baseline: 1.2673x; 1.0018x over previous
import jax
import jax.numpy as jnp
from jax import lax
from jax.experimental import pallas as pl
from jax.experimental.pallas import tpu as pltpu
from jax.experimental.pallas import tpu_sc as plsc

N = 10000
E = 320000
H = 128
HALF = 64
NC = 2
NS = 16
EB = 64
TCH = E // EB
NSTEP = (TCH + NS - 1) // NS + ((TCH + NS - 1) // NS) % 2
NPAIR = NSTEP // 2
ZB = 40
NZCHUNK = N // ZB
NZROUND = (NZCHUNK + NS - 1) // NS
RB = 200
NRCHUNK = N // RB
NRROUND = (NRCHUNK + NS - 1) // NS

_f32 = jnp.float32


def _sc_edge_body(xin_hbm, ea_hbm, src_hbm, dst_hbm, t_hbm, u_hbm, acc_hbm,
                  acc_sh, srcb0, srcb1, dstb0, dstb1, xsb0, xsb1, eab0, eab1,
                  valb0, valb1, tb, ub, ssrc0, ssrc1, sdat0, sdat1,
                  sdst0, sdst1, sscat0, sscat1):
    srcb = (srcb0, srcb1)
    dstb = (dstb0, dstb1)
    xsb = (xsb0, xsb1)
    eab = (eab0, eab1)
    valb = (valb0, valb1)
    ssrc = (ssrc0, ssrc1)
    sdat = (sdat0, sdat1)
    sdst = (sdst0, sdst1)
    sscat = (sscat0, sscat1)
    c = lax.axis_index("c")
    s = lax.axis_index("s")

    pltpu.sync_copy(t_hbm, tb)
    pltpu.sync_copy(u_hbm, ub)
    tv = tb[...]
    uv = ub[...]

    zero16 = jnp.zeros((16,), _f32)

    def _zrow(r, _):
        for q in range(2 * HALF // 16):
            valb0[r, pl.ds(16 * q, 16)] = zero16
        return 0

    lax.fori_loop(0, ZB, _zrow, 0)
    for k in range(NZROUND):
        cid = s + NS * k

        @pl.when(cid < NZCHUNK)
        def _():
            pltpu.sync_copy(valb0.at[pl.ds(0, ZB)],
                            acc_sh.at[pl.ds(cid * ZB, ZB)])
    plsc.subcore_barrier()

    c64 = c * HALF
    cEh = c * (E // 2)

    def _e0(jj):
        return (s + NS * jj) * EB

    def _valid(jj):
        return s + NS * jj < TCH

    def issue_src(jj, p):
        pltpu.async_copy(src_hbm.at[pl.ds(_e0(jj), EB)], srcb[p], ssrc[p])

    def wait_src(p):
        pltpu.make_async_copy(src_hbm.at[pl.ds(0, EB)], srcb[p], ssrc[p]).wait()

    def issue_dat(jj, p):
        pltpu.async_copy(xin_hbm.at[srcb[p]], xsb[p], sdat[p])
        pltpu.async_copy(
            ea_hbm.at[pl.ds(cEh + (s + NS * jj) * (EB // 2), EB // 2)],
            eab[p], sdat[p])

    def wait_dat(p):
        pltpu.make_async_copy(xin_hbm.at[srcb[p]], xsb[p], sdat[p]).wait()
        pltpu.make_async_copy(ea_hbm.at[pl.ds(0, EB // 2)], eab[p],
                              sdat[p]).wait()

    def wait_scat(p):
        pltpu.make_async_copy(valb[p], acc_sh.at[dstb[p]], sscat[p]).wait()

    def step(jj, p):
        q = 1 - p

        @pl.when(_valid(jj + 1))
        def _():
            wait_src(q)
            issue_dat(jj + 1, q)

        @pl.when(_valid(jj))
        def _():
            wait_dat(p)

            @pl.when(_valid(jj + 2))
            def _():
                issue_src(jj + 2, p)

            @pl.when(jj >= 2)
            def _():
                wait_scat(p)

            pltpu.async_copy(dst_hbm.at[pl.ds(_e0(jj), EB)], dstb[p], sdst[p])

            @plsc.parallel_loop(0, EB // 2, unroll=2)
            def _pair_edges(k):
                for par in range(2):
                    e = 2 * k + par
                    for qq in range(HALF // 16):
                        xq = xsb[p][e, pl.ds(c64 + 16 * qq, 16)]
                        aq = eab[p][k, pl.ds(par * HALF + 16 * qq, 16)]
                        msg = jnp.maximum(xq + aq, 0.0) + 1e-7
                        pw = jnp.exp(msg * tv - uv)
                        valb[p][e, pl.ds(16 * qq, 16)] = msg * pw
                        valb[p][e, pl.ds(HALF + 16 * qq, 16)] = pw

            pltpu.make_async_copy(dst_hbm.at[pl.ds(0, EB)], dstb[p],
                                  sdst[p]).wait()
            pltpu.async_copy(valb[p], acc_sh.at[dstb[p]], sscat[p], add=True)

    issue_src(0, 0)
    issue_src(1, 1)
    wait_src(0)
    issue_dat(0, 0)

    def _pair(i, _):
        step(2 * i, 0)
        step(2 * i + 1, 1)
        return 0

    lax.fori_loop(0, NPAIR, _pair, 0)
    wait_scat(0)
    wait_scat(1)
    plsc.subcore_barrier()

    cN = c * N
    for k in range(NRROUND):
        cid = s + NS * k

        @pl.when(cid < NRCHUNK)
        def _():
            r0 = cid * RB
            pltpu.sync_copy(acc_sh.at[pl.ds(r0, RB)],
                            acc_hbm.at[pl.ds(cN + r0, RB)])


_sc_edge = pl.kernel(
    _sc_edge_body,
    out_type=jax.ShapeDtypeStruct((NC * N, 2 * HALF), _f32),
    mesh=plsc.VectorSubcoreMesh(core_axis_name="c", subcore_axis_name="s"),
    scratch_types=[
        pltpu.VMEM_SHARED((N, 2 * HALF), _f32),
        pltpu.VMEM((EB,), jnp.int32),
        pltpu.VMEM((EB,), jnp.int32),
        pltpu.VMEM((EB,), jnp.int32),
        pltpu.VMEM((EB,), jnp.int32),
        pltpu.VMEM((EB, H), _f32),
        pltpu.VMEM((EB, H), _f32),
        pltpu.VMEM((EB // 2, H), _f32),
        pltpu.VMEM((EB // 2, H), _f32),
        pltpu.VMEM((EB, 2 * HALF), _f32),
        pltpu.VMEM((EB, 2 * HALF), _f32),
        pltpu.VMEM((16,), _f32),
        pltpu.VMEM((16,), _f32),
        pltpu.SemaphoreType.DMA,
        pltpu.SemaphoreType.DMA,
        pltpu.SemaphoreType.DMA,
        pltpu.SemaphoreType.DMA,
        pltpu.SemaphoreType.DMA,
        pltpu.SemaphoreType.DMA,
        pltpu.SemaphoreType.DMA,
        pltpu.SemaphoreType.DMA,
    ],
)


def _acc_max(mx_ref, m):
    i = pl.program_id(0)

    @pl.when(i == 0)
    def _():
        mx_ref[0, 0] = m

    @pl.when(i > 0)
    def _():
        mx_ref[0, 0] = jnp.maximum(mx_ref[0, 0], m)


def _k_edge_enc(a2_ref, wlo_ref, whi_ref, blo_ref, bhi_ref, out_ref, mx_ref):
    a2 = a2_ref[...]
    lo = jnp.dot(a2, wlo_ref[...], preferred_element_type=_f32) + blo_ref[...]
    hi = jnp.dot(a2, whi_ref[...], preferred_element_type=_f32) + bhi_ref[...]
    out_ref[0] = lo
    out_ref[1] = hi
    _acc_max(mx_ref, jnp.maximum(jnp.max(lo), jnp.max(hi)))


def _k_node_enc(x_ref, w_ref, b_ref, out_ref, mx_ref):
    y = jnp.dot(x_ref[...], w_ref[...], preferred_element_type=_f32) + b_ref[...]
    out_ref[...] = y
    _acc_max(mx_ref, jnp.max(y))


def _ln_rows(x, g, b):
    mu = jnp.mean(x, axis=-1, keepdims=True)
    xc = x - mu
    var = jnp.mean(xc * xc, axis=-1, keepdims=True)
    return xc * jax.lax.rsqrt(var + 1e-5) * g + b


def _k_norm_act(x_ref, g_ref, b_ref, out_ref, mx_ref):
    y = jnp.maximum(_ln_rows(x_ref[...], g_ref[...], b_ref[...]), 0.0)
    out_ref[...] = y
    _acc_max(mx_ref, jnp.max(y))


def _k_mlp(xin_ref, acc_ref, xprev_ref, w1_ref, b1_ref, g1_ref, be1_ref,
           w2_ref, b2_ref, out_ref):
    a0 = acc_ref[0]
    a1 = acc_ref[1]
    lo = jnp.where(a0[:, HALF:] > 0.0, a0[:, :HALF] / a0[:, HALF:], 0.0)
    hi = jnp.where(a1[:, HALF:] > 0.0, a1[:, :HALF] / a1[:, HALF:], 0.0)
    h = xin_ref[...] + jnp.concatenate([lo, hi], axis=-1)
    z = jnp.dot(h, w1_ref[...], preferred_element_type=_f32) + b1_ref[...]
    z = jnp.maximum(_ln_rows(z, g1_ref[...], be1_ref[...]), 0.0)
    z = jnp.dot(z, w2_ref[...], preferred_element_type=_f32) + b2_ref[...]
    out_ref[...] = xprev_ref[...] + z


def _k_final(x_ref, g_ref, b_ref, w_ref, bo_ref, out_ref):
    y = jnp.maximum(_ln_rows(x_ref[...], g_ref[...], b_ref[...]), 0.0)
    out_ref[...] = jnp.dot(y, w_ref[...], preferred_element_type=_f32) + bo_ref[...]


def _full(shape):
    nd = len(shape)
    return pl.BlockSpec(shape, lambda i, _nd=nd: (0,) * _nd)


def _enc_max_call(body, x, w, b, bn):
    n = x.shape[0]
    return pl.pallas_call(
        body,
        grid=(n // bn,),
        in_specs=[
            pl.BlockSpec((bn, x.shape[1]), lambda i: (i, 0)),
            _full(w.shape),
            _full(b.shape),
        ],
        out_specs=[
            pl.BlockSpec((bn, H), lambda i: (i, 0)),
            pl.BlockSpec(memory_space=pltpu.SMEM),
        ],
        out_shape=[
            jax.ShapeDtypeStruct((n, H), _f32),
            jax.ShapeDtypeStruct((1, 1), _f32),
        ],
    )(x, w, b)


BN = 2000
BE_ENC = 4000


def kernel(x, edge_index, edge_attr, params):
    src = edge_index[0]
    dst = edge_index[1]

    We, be = params['edge_enc']
    D_E = edge_attr.shape[1]
    attr2 = edge_attr.reshape(E // 2, 2 * D_E)
    zpad = jnp.zeros((D_E, HALF), _f32)
    wlo = jnp.concatenate(
        [jnp.concatenate([We[:, :HALF], zpad], axis=1),
         jnp.concatenate([zpad, We[:, :HALF]], axis=1)], axis=0)
    whi = jnp.concatenate(
        [jnp.concatenate([We[:, HALF:], zpad], axis=1),
         jnp.concatenate([zpad, We[:, HALF:]], axis=1)], axis=0)
    blo = jnp.concatenate([be[:HALF], be[:HALF]]).reshape(1, H)
    bhi = jnp.concatenate([be[HALF:], be[HALF:]]).reshape(1, H)
    BE2 = 2000
    ea_pack, mx_ea = pl.pallas_call(
        _k_edge_enc,
        grid=(E // 2 // BE2,),
        in_specs=[
            pl.BlockSpec((BE2, 2 * D_E), lambda i: (i, 0)),
            _full((2 * D_E, H)), _full((2 * D_E, H)),
            _full((1, H)), _full((1, H)),
        ],
        out_specs=[
            pl.BlockSpec((2, BE2, H), lambda i: (0, i, 0)),
            pl.BlockSpec(memory_space=pltpu.SMEM),
        ],
        out_shape=[
            jax.ShapeDtypeStruct((2, E // 2, H), _f32),
            jax.ShapeDtypeStruct((1, 1), _f32),
        ],
    )(attr2, wlo, whi, blo, bhi)
    ea = ea_pack.reshape(E, H)

    Wn, bn_ = params['node_enc']
    xin, mx_x = _enc_max_call(_k_node_enc, x, Wn, bn_.reshape(1, H), BN)

    mlp_call = pl.pallas_call(
        _k_mlp,
        grid=(N // BN,),
        in_specs=[
            pl.BlockSpec((BN, H), lambda i: (i, 0)),
            pl.BlockSpec((2, BN, H), lambda i: (0, i, 0)),
            pl.BlockSpec((BN, H), lambda i: (i, 0)),
            _full((H, 2 * H)), _full((1, 2 * H)), _full((1, 2 * H)),
            _full((1, 2 * H)), _full((2 * H, H)), _full((1, H)),
        ],
        out_specs=pl.BlockSpec((BN, H), lambda i: (i, 0)),
        out_shape=jax.ShapeDtypeStruct((N, H), _f32),
    )

    x_run = jnp.zeros((N, H), _f32)
    for li, lp in enumerate(params['layers']):
        if li > 0:
            g, bb = lp['norm']
            xin, mx_x = _enc_max_call(
                _k_norm_act, x_run, g.reshape(1, H), bb.reshape(1, H), BN)
        t = lp['t']
        u = t * (jnp.maximum(mx_x[0, 0] + mx_ea[0, 0], 0.0) + 1e-7)
        t16 = jnp.broadcast_to(t.astype(_f32), (16,))
        u16 = jnp.broadcast_to(u.astype(_f32), (16,))
        acc_flat = _sc_edge(xin, ea, src, dst, t16, u16)
        acc2 = acc_flat.reshape(NC, N, 2 * HALF)
        W1, b1 = lp['mlp_w1']
        g1, be1 = lp['mlp_ln']
        W2, b2 = lp['mlp_w2']
        x_run = mlp_call(xin, acc2, x_run, W1, b1.reshape(1, 2 * H),
                         g1.reshape(1, 2 * H), be1.reshape(1, 2 * H),
                         W2, b2.reshape(1, H))

    g0, b0 = params['layers'][0]['norm']
    Wo, bo = params['lin_out']
    out = pl.pallas_call(
        _k_final,
        grid=(N // BN,),
        in_specs=[
            pl.BlockSpec((BN, H), lambda i: (i, 0)),
            _full((1, H)), _full((1, H)), _full((H, H)), _full((1, H)),
        ],
        out_specs=pl.BlockSpec((BN, H), lambda i: (i, 0)),
        out_shape=jax.ShapeDtypeStruct((N, H), _f32),
    )(x_run, g0.reshape(1, H), b0.reshape(1, H), Wo, bo.reshape(1, H))
    return out


# speedup vs baseline: 1.3122x; 1.0354x over previous
import jax
import jax.numpy as jnp
from jax import lax
from jax.experimental import pallas as pl
from jax.experimental.pallas import tpu as pltpu
from jax.experimental.pallas import tpu_sc as plsc

N = 10000
E = 320000
H = 128
HALF = 64
NUM_L = 4
NC = 2
NS = 16
EB = 64
TCH = E // EB
NSTEP = (TCH + NS - 1) // NS + ((TCH + NS - 1) // NS) % 2
NPAIR = NSTEP // 2
ZB = 40
NZCHUNK = N // ZB
NZROUND = (NZCHUNK + NS - 1) // NS
RB = 200
NRCHUNK = N // RB
NRROUND = (NRCHUNK + NS - 1) // NS

_f32 = jnp.float32


def _sc_edge_body(xin_hbm, ea_hbm, src_hbm, dst_hbm, t_hbm, u_hbm, acc_hbm,
                  acc_sh, srcb0, srcb1, dstb0, dstb1, xsb0, xsb1, eab0, eab1,
                  valb0, valb1, tb, ub, ssrc0, ssrc1, sdat0, sdat1,
                  sdst0, sdst1, sscat0, sscat1):
    srcb = (srcb0, srcb1)
    dstb = (dstb0, dstb1)
    xsb = (xsb0, xsb1)
    eab = (eab0, eab1)
    valb = (valb0, valb1)
    ssrc = (ssrc0, ssrc1)
    sdat = (sdat0, sdat1)
    sdst = (sdst0, sdst1)
    sscat = (sscat0, sscat1)
    c = lax.axis_index("c")
    s = lax.axis_index("s")

    pltpu.sync_copy(t_hbm, tb)
    pltpu.sync_copy(u_hbm, ub)
    tv = tb[...]
    uv = ub[...]

    zero16 = jnp.zeros((16,), _f32)

    def _zrow(r, _):
        for q in range(2 * HALF // 16):
            valb0[r, pl.ds(16 * q, 16)] = zero16
        return 0

    lax.fori_loop(0, ZB, _zrow, 0)
    for k in range(NZROUND):
        cid = s + NS * k

        @pl.when(cid < NZCHUNK)
        def _():
            pltpu.sync_copy(valb0.at[pl.ds(0, ZB)],
                            acc_sh.at[pl.ds(cid * ZB, ZB)])
    plsc.subcore_barrier()

    c64 = c * HALF
    cEh = c * (E // 2)

    def _e0(jj):
        return (s + NS * jj) * EB

    def _valid(jj):
        return s + NS * jj < TCH

    def issue_src(jj, p):
        pltpu.async_copy(src_hbm.at[pl.ds(_e0(jj), EB)], srcb[p], ssrc[p])

    def wait_src(p):
        pltpu.make_async_copy(src_hbm.at[pl.ds(0, EB)], srcb[p], ssrc[p]).wait()

    def issue_dat(jj, p):
        pltpu.async_copy(xin_hbm.at[srcb[p]], xsb[p], sdat[p])
        pltpu.async_copy(
            ea_hbm.at[pl.ds(cEh + (s + NS * jj) * (EB // 2), EB // 2)],
            eab[p], sdat[p])

    def wait_dat(p):
        pltpu.make_async_copy(xin_hbm.at[srcb[p]], xsb[p], sdat[p]).wait()
        pltpu.make_async_copy(ea_hbm.at[pl.ds(0, EB // 2)], eab[p],
                              sdat[p]).wait()

    def wait_scat(p):
        pltpu.make_async_copy(valb[p], acc_sh.at[dstb[p]], sscat[p]).wait()

    def step(jj, p):
        q = 1 - p

        @pl.when(_valid(jj + 1))
        def _():
            wait_src(q)
            issue_dat(jj + 1, q)

        @pl.when(_valid(jj))
        def _():
            wait_dat(p)

            @pl.when(_valid(jj + 2))
            def _():
                issue_src(jj + 2, p)

            @pl.when(jj >= 2)
            def _():
                wait_scat(p)

            pltpu.async_copy(dst_hbm.at[pl.ds(_e0(jj), EB)], dstb[p], sdst[p])

            @plsc.parallel_loop(0, EB // 2, unroll=2)
            def _pair_edges(k):
                for par in range(2):
                    e = 2 * k + par
                    for qq in range(HALF // 16):
                        xq = xsb[p][e, pl.ds(c64 + 16 * qq, 16)]
                        aq = eab[p][k, pl.ds(par * HALF + 16 * qq, 16)]
                        msg = jnp.maximum(xq + aq, 0.0) + 1e-7
                        pw = jnp.exp(msg * tv - uv)
                        valb[p][e, pl.ds(16 * qq, 16)] = msg * pw
                        valb[p][e, pl.ds(HALF + 16 * qq, 16)] = pw

            pltpu.make_async_copy(dst_hbm.at[pl.ds(0, EB)], dstb[p],
                                  sdst[p]).wait()
            pltpu.async_copy(valb[p], acc_sh.at[dstb[p]], sscat[p], add=True)

    issue_src(0, 0)
    issue_src(1, 1)
    wait_src(0)
    issue_dat(0, 0)

    def _pair(i, _):
        step(2 * i, 0)
        step(2 * i + 1, 1)
        return 0

    lax.fori_loop(0, NPAIR, _pair, 0)
    wait_scat(0)
    wait_scat(1)
    plsc.subcore_barrier()

    cN = c * N
    for k in range(NRROUND):
        cid = s + NS * k

        @pl.when(cid < NRCHUNK)
        def _():
            r0 = cid * RB
            pltpu.sync_copy(acc_sh.at[pl.ds(r0, RB)],
                            acc_hbm.at[pl.ds(cN + r0, RB)])


_sc_edge = pl.kernel(
    _sc_edge_body,
    out_type=jax.ShapeDtypeStruct((NC * N, 2 * HALF), _f32),
    mesh=plsc.VectorSubcoreMesh(core_axis_name="c", subcore_axis_name="s"),
    scratch_types=[
        pltpu.VMEM_SHARED((N, 2 * HALF), _f32),
        pltpu.VMEM((EB,), jnp.int32),
        pltpu.VMEM((EB,), jnp.int32),
        pltpu.VMEM((EB,), jnp.int32),
        pltpu.VMEM((EB,), jnp.int32),
        pltpu.VMEM((EB, H), _f32),
        pltpu.VMEM((EB, H), _f32),
        pltpu.VMEM((EB // 2, H), _f32),
        pltpu.VMEM((EB // 2, H), _f32),
        pltpu.VMEM((EB, 2 * HALF), _f32),
        pltpu.VMEM((EB, 2 * HALF), _f32),
        pltpu.VMEM((16,), _f32),
        pltpu.VMEM((16,), _f32),
        pltpu.SemaphoreType.DMA,
        pltpu.SemaphoreType.DMA,
        pltpu.SemaphoreType.DMA,
        pltpu.SemaphoreType.DMA,
        pltpu.SemaphoreType.DMA,
        pltpu.SemaphoreType.DMA,
        pltpu.SemaphoreType.DMA,
        pltpu.SemaphoreType.DMA,
    ],
)


def _acc_max(mx_ref, m):
    i = pl.program_id(0)

    @pl.when(i == 0)
    def _():
        mx_ref[0, 0] = m

    @pl.when(i > 0)
    def _():
        mx_ref[0, 0] = jnp.maximum(mx_ref[0, 0], m)


def _k_edge_enc(a2_ref, wlo_ref, whi_ref, blo_ref, bhi_ref, out_ref, mx_ref):
    a2 = a2_ref[...]
    lo = jnp.dot(a2, wlo_ref[...], preferred_element_type=_f32) + blo_ref[...]
    hi = jnp.dot(a2, whi_ref[...], preferred_element_type=_f32) + bhi_ref[...]
    out_ref[0] = lo
    out_ref[1] = hi
    _acc_max(mx_ref, jnp.maximum(jnp.max(lo), jnp.max(hi)))


def _k_node_enc(x_ref, w_ref, b_ref, out_ref, mx_ref):
    y = jnp.dot(x_ref[...], w_ref[...], preferred_element_type=_f32) + b_ref[...]
    out_ref[...] = y
    _acc_max(mx_ref, jnp.max(y))


def _ln_rows(x, g, b):
    mu = jnp.mean(x, axis=-1, keepdims=True)
    xc = x - mu
    var = jnp.mean(xc * xc, axis=-1, keepdims=True)
    return xc * jax.lax.rsqrt(var + 1e-5) * g + b


def _k_mlp(xin_ref, acc_ref, xprev_ref, w1_ref, b1_ref, g1_ref, be1_ref,
           w2_ref, b2_ref, gn_ref, bn_ref, out_ref, nxt_ref, mx_ref):
    a0 = acc_ref[0]
    a1 = acc_ref[1]
    lo = jnp.where(a0[:, HALF:] > 0.0, a0[:, :HALF] / a0[:, HALF:], 0.0)
    hi = jnp.where(a1[:, HALF:] > 0.0, a1[:, :HALF] / a1[:, HALF:], 0.0)
    h = xin_ref[...] + jnp.concatenate([lo, hi], axis=-1)
    z = jnp.dot(h, w1_ref[...], preferred_element_type=_f32) + b1_ref[...]
    z = jnp.maximum(_ln_rows(z, g1_ref[...], be1_ref[...]), 0.0)
    z = jnp.dot(z, w2_ref[...], preferred_element_type=_f32) + b2_ref[...]
    xnew = xprev_ref[...] + z
    out_ref[...] = xnew
    y = jnp.maximum(_ln_rows(xnew, gn_ref[...], bn_ref[...]), 0.0)
    nxt_ref[...] = y
    _acc_max(mx_ref, jnp.max(y))


def _k_final(x_ref, g_ref, b_ref, w_ref, bo_ref, out_ref):
    y = jnp.maximum(_ln_rows(x_ref[...], g_ref[...], b_ref[...]), 0.0)
    out_ref[...] = jnp.dot(y, w_ref[...], preferred_element_type=_f32) + bo_ref[...]


def _full(shape):
    nd = len(shape)
    return pl.BlockSpec(shape, lambda i, _nd=nd: (0,) * _nd)


def _enc_max_call(body, x, w, b, bn):
    n = x.shape[0]
    return pl.pallas_call(
        body,
        grid=(n // bn,),
        in_specs=[
            pl.BlockSpec((bn, x.shape[1]), lambda i: (i, 0)),
            _full(w.shape),
            _full(b.shape),
        ],
        out_specs=[
            pl.BlockSpec((bn, H), lambda i: (i, 0)),
            pl.BlockSpec(memory_space=pltpu.SMEM),
        ],
        out_shape=[
            jax.ShapeDtypeStruct((n, H), _f32),
            jax.ShapeDtypeStruct((1, 1), _f32),
        ],
    )(x, w, b)


BN = 2000
BE_ENC = 4000


def kernel(x, edge_index, edge_attr, params):
    src = edge_index[0]
    dst = edge_index[1]

    We, be = params['edge_enc']
    D_E = edge_attr.shape[1]
    attr2 = edge_attr.reshape(E // 2, 2 * D_E)
    zpad = jnp.zeros((D_E, HALF), _f32)
    wlo = jnp.concatenate(
        [jnp.concatenate([We[:, :HALF], zpad], axis=1),
         jnp.concatenate([zpad, We[:, :HALF]], axis=1)], axis=0)
    whi = jnp.concatenate(
        [jnp.concatenate([We[:, HALF:], zpad], axis=1),
         jnp.concatenate([zpad, We[:, HALF:]], axis=1)], axis=0)
    blo = jnp.concatenate([be[:HALF], be[:HALF]]).reshape(1, H)
    bhi = jnp.concatenate([be[HALF:], be[HALF:]]).reshape(1, H)
    BE2 = 2000
    ea_pack, mx_ea = pl.pallas_call(
        _k_edge_enc,
        grid=(E // 2 // BE2,),
        in_specs=[
            pl.BlockSpec((BE2, 2 * D_E), lambda i: (i, 0)),
            _full((2 * D_E, H)), _full((2 * D_E, H)),
            _full((1, H)), _full((1, H)),
        ],
        out_specs=[
            pl.BlockSpec((2, BE2, H), lambda i: (0, i, 0)),
            pl.BlockSpec(memory_space=pltpu.SMEM),
        ],
        out_shape=[
            jax.ShapeDtypeStruct((2, E // 2, H), _f32),
            jax.ShapeDtypeStruct((1, 1), _f32),
        ],
    )(attr2, wlo, whi, blo, bhi)
    ea = ea_pack.reshape(E, H)

    Wn, bn_ = params['node_enc']
    xin, mx_x = _enc_max_call(_k_node_enc, x, Wn, bn_.reshape(1, H), BN)

    mlp_call = pl.pallas_call(
        _k_mlp,
        grid=(N // BN,),
        in_specs=[
            pl.BlockSpec((BN, H), lambda i: (i, 0)),
            pl.BlockSpec((2, BN, H), lambda i: (0, i, 0)),
            pl.BlockSpec((BN, H), lambda i: (i, 0)),
            _full((H, 2 * H)), _full((1, 2 * H)), _full((1, 2 * H)),
            _full((1, 2 * H)), _full((2 * H, H)), _full((1, H)),
            _full((1, H)), _full((1, H)),
        ],
        out_specs=[
            pl.BlockSpec((BN, H), lambda i: (i, 0)),
            pl.BlockSpec((BN, H), lambda i: (i, 0)),
            pl.BlockSpec(memory_space=pltpu.SMEM),
        ],
        out_shape=[
            jax.ShapeDtypeStruct((N, H), _f32),
            jax.ShapeDtypeStruct((N, H), _f32),
            jax.ShapeDtypeStruct((1, 1), _f32),
        ],
    )

    layers = params['layers']
    x_run = jnp.zeros((N, H), _f32)
    for li, lp in enumerate(layers):
        t = lp['t']
        u = t * (jnp.maximum(mx_x[0, 0] + mx_ea[0, 0], 0.0) + 1e-7)
        t16 = jnp.broadcast_to(t.astype(_f32), (16,))
        u16 = jnp.broadcast_to(u.astype(_f32), (16,))
        acc_flat = _sc_edge(xin, ea, src, dst, t16, u16)
        acc2 = acc_flat.reshape(NC, N, 2 * HALF)
        W1, b1 = lp['mlp_w1']
        g1, be1 = lp['mlp_ln']
        W2, b2 = lp['mlp_w2']
        gn, bnn = layers[li + 1 if li + 1 < NUM_L else 0]['norm']
        x_run, xin, mx_x = mlp_call(
            xin, acc2, x_run, W1, b1.reshape(1, 2 * H),
            g1.reshape(1, 2 * H), be1.reshape(1, 2 * H),
            W2, b2.reshape(1, H), gn.reshape(1, H), bnn.reshape(1, H))

    g0, b0 = params['layers'][0]['norm']
    Wo, bo = params['lin_out']
    out = pl.pallas_call(
        _k_final,
        grid=(N // BN,),
        in_specs=[
            pl.BlockSpec((BN, H), lambda i: (i, 0)),
            _full((1, H)), _full((1, H)), _full((H, H)), _full((1, H)),
        ],
        out_specs=pl.BlockSpec((BN, H), lambda i: (i, 0)),
        out_shape=jax.ShapeDtypeStruct((N, H), _f32),
    )(x_run, g0.reshape(1, H), b0.reshape(1, H), Wo, bo.reshape(1, H))
    return out
